# Initial kernel scaffold; baseline (speedup 1.0000x reference)
import jax
import jax.numpy as jnp
from jax.experimental import pallas as pl


def kernel(query, key, value, reference_points, spatial_shapes, reference_points_cam, bev_mask, W_off, b_off, W_attn, b_attn, W_v, b_v, W_do, b_do, W_o, b_o):
    raise NotImplementedError("write your pallas kernel here")


# speedup vs baseline: 47.4311x; 47.4311x over previous
import functools

import jax
import jax.numpy as jnp
from jax.experimental import pallas as pl
from jax.experimental.pallas import tpu as pltpu

EMBED = 256
HEADS = 8
DH = EMBED // HEADS
LEVELS = 4
POINTS = 4
CAMS = 6
NQ = 10000
SHAPES = [(46, 80), (23, 40), (12, 20), (6, 10)]
NK = sum(h * w for h, w in SHAPES)
STARTS = [0, 3680, 4600, 4840]


def _final_body(agg_ref, cnt_ref, q_ref, wdo_ref, bdo_ref, wo_ref, bo_ref, out_ref):
    cnt = cnt_ref[...]
    has = (cnt > 0.0).astype(jnp.float32)
    x = agg_ref[...] / jnp.maximum(cnt, 1.0)
    s = jnp.dot(x, wdo_ref[...], preferred_element_type=jnp.float32) + has * bdo_ref[...]
    o = jnp.dot(s, wo_ref[...], preferred_element_type=jnp.float32) + bo_ref[...]
    out_ref[...] = o + q_ref[...]


def _final_call(agg, cnt, q2, W_do, b_do, W_o, b_o):
    blk = 2000
    grid = (NQ // blk,)
    return pl.pallas_call(
        _final_body,
        grid=grid,
        in_specs=[
            pl.BlockSpec((blk, EMBED), lambda i: (i, 0)),
            pl.BlockSpec((blk, 1), lambda i: (i, 0)),
            pl.BlockSpec((blk, EMBED), lambda i: (i, 0)),
            pl.BlockSpec((EMBED, EMBED), lambda i: (0, 0)),
            pl.BlockSpec((1, EMBED), lambda i: (0, 0)),
            pl.BlockSpec((EMBED, EMBED), lambda i: (0, 0)),
            pl.BlockSpec((1, EMBED), lambda i: (0, 0)),
        ],
        out_specs=pl.BlockSpec((blk, EMBED), lambda i: (i, 0)),
        out_shape=jax.ShapeDtypeStruct((NQ, EMBED), jnp.float32),
    )(agg, cnt, q2, W_do, b_do[None, :], W_o, b_o[None, :])


def kernel(query, key, value, reference_points, spatial_shapes, reference_points_cam,
           bev_mask, W_off, b_off, W_attn, b_attn, W_v, b_v, W_do, b_do, W_o, b_o):
    del key, reference_points, spatial_shapes
    q2 = query[0]
    off = (q2 @ W_off + b_off).reshape(NQ, HEADS, LEVELS, POINTS, 2)
    logits = (q2 @ W_attn + b_attn).reshape(NQ, HEADS, LEVELS * POINTS)
    aw = jax.nn.softmax(logits, axis=-1).reshape(NQ, HEADS, LEVELS, POINTS)
    vp = (value[:, 0] @ W_v + b_v).reshape(CAMS, NK, HEADS, DH)
    active = (bev_mask[:, 0].sum(-1) > 0).astype(jnp.float32)
    rcam = reference_points_cam[:, 0]

    agg = jnp.zeros((NQ, HEADS, DH), jnp.float32)
    for c in range(CAMS):
        acc = jnp.zeros((NQ, HEADS, DH), jnp.float32)
        for l, (Hl, Wl) in enumerate(SHAPES):
            vph = vp[c, STARTS[l]:STARTS[l] + Hl * Wl].transpose(1, 0, 2)
            x = rcam[c, :, None, l, 0, None] * Wl + off[:, :, l, :, 0] - 0.5
            y = rcam[c, :, None, l, 1, None] * Hl + off[:, :, l, :, 1] - 0.5
            x0 = jnp.floor(x); y0 = jnp.floor(y)
            wx1 = x - x0; wy1 = y - y0
            samp = jnp.zeros((NQ, HEADS, POINTS, DH), jnp.float32)
            for dx, dy, w in ((0, 0, (1 - wx1) * (1 - wy1)), (1, 0, wx1 * (1 - wy1)),
                              (0, 1, (1 - wx1) * wy1), (1, 1, wx1 * wy1)):
                xi = x0 + dx; yi = y0 + dy
                valid = ((xi >= 0) & (xi <= Wl - 1) & (yi >= 0) & (yi <= Hl - 1)).astype(jnp.float32)
                xc = jnp.clip(xi, 0, Wl - 1).astype(jnp.int32)
                yc = jnp.clip(yi, 0, Hl - 1).astype(jnp.int32)
                idx = (yc * Wl + xc).transpose(1, 0, 2).reshape(HEADS, NQ * POINTS)
                v = jnp.take_along_axis(vph, idx[:, :, None], axis=1)
                v = v.reshape(HEADS, NQ, POINTS, DH).transpose(1, 0, 2, 3)
                samp = samp + v * (w * valid)[:, :, :, None]
            acc = acc + (samp * aw[:, :, l, :, None]).sum(2)
        agg = agg + acc * active[c][:, None, None]

    cnt = active.sum(0)[:, None]
    out = _final_call(agg.reshape(NQ, EMBED), cnt, q2, W_do, b_do, W_o, b_o)
    return out[None]


# speedup vs baseline: 823.7134x; 17.3665x over previous
import functools

import jax
import jax.numpy as jnp
from jax import lax
from jax.experimental import pallas as pl
from jax.experimental.pallas import tpu as pltpu
from jax.experimental.pallas import tpu_sc as plsc

EMBED = 256
HEADS = 8
DH = EMBED // HEADS
LEVELS = 4
POINTS = 4
CAMS = 6
NQ = 10000
SHAPES = [(46, 80), (23, 40), (12, 20), (6, 10)]
NK = sum(h * w for h, w in SHAPES)
STARTS = [0, 3680, 4600, 4840]
NQH = NQ * HEADS
CONTRIB = CAMS * LEVELS * POINTS * 4
NROWS = CAMS * NK * HEADS

NTILES = 32
QH_T = NQH // NTILES
CH = 2
CROWS = CH * CONTRIB
NSUB = CROWS // 128
NCH = QH_T // CH


def _vp_body(v_ref, w_ref, b_ref, out_ref):
    out_ref[...] = (jnp.dot(v_ref[...], w_ref[...],
                            preferred_element_type=jnp.float32) + b_ref[...])


def _vp_call(val2d, W_v, b_v):
    blk = 4200
    return pl.pallas_call(
        _vp_body,
        grid=(val2d.shape[0] // blk,),
        in_specs=[
            pl.BlockSpec((blk, EMBED), lambda i: (i, 0)),
            pl.BlockSpec((EMBED, EMBED), lambda i: (0, 0)),
            pl.BlockSpec((1, EMBED), lambda i: (0, 0)),
        ],
        out_specs=pl.BlockSpec((blk, EMBED), lambda i: (i, 0)),
        out_shape=jax.ShapeDtypeStruct((val2d.shape[0], EMBED), jnp.float32),
    )(val2d, W_v, b_v[None, :])


def _qproj_body(q_ref, woff_ref, boff_ref, wattn_ref, battn_ref, off_ref, aw_ref):
    q = q_ref[...]
    off_ref[...] = (jnp.dot(q, woff_ref[...], preferred_element_type=jnp.float32)
                    + boff_ref[...])
    lg = (jnp.dot(q, wattn_ref[...], preferred_element_type=jnp.float32)
          + battn_ref[...])
    lane = lax.broadcasted_iota(jnp.int32, lg.shape, 1)
    m = lg
    for sh in (8, 4, 2, 1):
        lo = (lane & sh) == 0
        m = jnp.maximum(m, jnp.where(lo, pltpu.roll(m, 128 - sh, axis=1),
                                     pltpu.roll(m, sh, axis=1)))
    e = jnp.exp(lg - m)
    s = e
    for sh in (8, 4, 2, 1):
        lo = (lane & sh) == 0
        s = s + jnp.where(lo, pltpu.roll(s, 128 - sh, axis=1),
                          pltpu.roll(s, sh, axis=1))
    aw_ref[...] = e / s


def _qproj_call(q2, W_off, b_off, W_attn, b_attn):
    blk = 2000
    return pl.pallas_call(
        _qproj_body,
        grid=(NQ // blk,),
        in_specs=[
            pl.BlockSpec((blk, EMBED), lambda i: (i, 0)),
            pl.BlockSpec((EMBED, EMBED), lambda i: (0, 0)),
            pl.BlockSpec((1, EMBED), lambda i: (0, 0)),
            pl.BlockSpec((EMBED, 128), lambda i: (0, 0)),
            pl.BlockSpec((1, 128), lambda i: (0, 0)),
        ],
        out_specs=[
            pl.BlockSpec((blk, EMBED), lambda i: (i, 0)),
            pl.BlockSpec((blk, 128), lambda i: (i, 0)),
        ],
        out_shape=[
            jax.ShapeDtypeStruct((NQ, EMBED), jnp.float32),
            jax.ShapeDtypeStruct((NQ, 128), jnp.float32),
        ],
    )(q2, W_off, b_off[None, :], W_attn, b_attn[None, :])


def _build_idx_w(off, aw, rcam, active):
    WlA = jnp.array([w for (h, w) in SHAPES], jnp.float32)
    HlA = jnp.array([h for (h, w) in SHAPES], jnp.float32)
    WlI = jnp.array([w for (h, w) in SHAPES], jnp.int32)
    startA = jnp.array(STARTS, jnp.int32)

    rc_x = rcam[..., 0].transpose(1, 0, 2)[:, None, :, :, None]
    rc_y = rcam[..., 1].transpose(1, 0, 2)[:, None, :, :, None]
    of_x = off[..., 0][:, :, None, :, :]
    of_y = off[..., 1][:, :, None, :, :]
    x = rc_x * WlA[None, None, None, :, None] + of_x - 0.5
    y = rc_y * HlA[None, None, None, :, None] + of_y - 0.5
    x0 = jnp.floor(x)
    y0 = jnp.floor(y)
    wx1 = x - x0
    wy1 = y - y0

    base_w = (aw[:, :, None, :, :] * active.T[:, None, :, None, None])
    cidx = lax.broadcasted_iota(jnp.int32, x.shape, 2)
    hidx = lax.broadcasted_iota(jnp.int32, x.shape, 1)
    WlIb = WlI[None, None, None, :, None]
    HlIb = jnp.array([h for (h, w) in SHAPES], jnp.int32)[None, None, None, :, None]
    startb = startA[None, None, None, :, None]

    idxs = []
    ws = []
    for dx, dy, cw in ((0, 0, (1 - wx1) * (1 - wy1)), (1, 0, wx1 * (1 - wy1)),
                       (0, 1, (1 - wx1) * wy1), (1, 1, wx1 * wy1)):
        xi = x0 + dx
        yi = y0 + dy
        valid = ((xi >= 0) & (xi <= WlIb.astype(jnp.float32) - 1)
                 & (yi >= 0) & (yi <= HlIb.astype(jnp.float32) - 1))
        xc = jnp.clip(xi, 0, WlIb.astype(jnp.float32) - 1).astype(jnp.int32)
        yc = jnp.clip(yi, 0, HlIb.astype(jnp.float32) - 1).astype(jnp.int32)
        pix = startb + yc * WlIb + xc
        row = (cidx * NK + pix) * HEADS + hidx
        idxs.append(row)
        ws.append(base_w * cw * valid.astype(jnp.float32))
    idx = jnp.stack(idxs, axis=-1).reshape(NQH, CONTRIB)
    w = jnp.stack(ws, axis=-1).reshape(NQH, CONTRIB)
    return idx, w


def _bag_body(idx_hbm, w_hbm, table_hbm, out_hbm,
              idxv0, idxv1, wv0, wv1, rows0, rows1, outv0, outv1,
              g0, g1, i0, i1, ws0, ws1, o0, o1):
    idxv = (idxv0, idxv1)
    wv = (wv0, wv1)
    rows = (rows0, rows1)
    outv = (outv0, outv1)
    gsem = (g0, g1)
    isem = (i0, i1)
    wsem = (ws0, ws1)
    osem = (o0, o1)

    wid = lax.axis_index("s") * 2 + lax.axis_index("c")
    qh0 = wid * QH_T

    def idx_slice(c):
        return idx_hbm.at[pl.ds((qh0 + c * CH) * CONTRIB, CROWS)]

    def w_slice(c):
        return w_hbm.at[pl.ds((qh0 + c * CH) * CONTRIB, CROWS)]

    def out_slice(c):
        return out_hbm.at[pl.ds((qh0 + c * CH) * DH, CH * DH)]

    def fire_gathers(p, c):
        for s in range(NSUB):
            pltpu.async_copy(
                table_hbm.at[idxv[p].at[pl.ds(s * 128, 128)]],
                rows[p].at[pl.ds(s * 128, 128), :], gsem[p])

    def drain_gathers(p):
        for s in range(NSUB):
            pltpu.make_async_copy(
                table_hbm.at[idxv[p].at[pl.ds(s * 128, 128)]],
                rows[p].at[pl.ds(s * 128, 128), :], gsem[p]).wait()

    iota16 = lax.broadcasted_iota(jnp.int32, (16,), 0)

    def compute(p, c):
        for j in range(CH):
            jbase = j * CONTRIB

            def g_body(g, accs):
                acc0, acc1 = accs
                k0 = jbase + g * 16
                wv16 = wv[p][pl.ds(k0, 16)]
                for t in range(16):
                    wsplat = jnp.take_along_axis(
                        wv16, jnp.full((16,), t, jnp.int32), axis=0)
                    r0 = rows[p][k0 + t, pl.ds(0, 16)]
                    r1 = rows[p][k0 + t, pl.ds(16, 16)]
                    acc0 = acc0 + wsplat * r0
                    acc1 = acc1 + wsplat * r1
                return acc0, acc1

            z = jnp.zeros((16,), jnp.float32)
            acc0, acc1 = lax.fori_loop(0, CONTRIB // 16, g_body, (z, z))
            outv[p][pl.ds(j * DH, 16)] = acc0
            outv[p][pl.ds(j * DH + 16, 16)] = acc1

    pltpu.sync_copy(idx_slice(0), idxv[0])
    fire_gathers(0, 0)
    pltpu.async_copy(idx_slice(1), idxv[1], isem[1])
    pltpu.async_copy(w_slice(0), wv[0], wsem[0])
    pltpu.async_copy(w_slice(1), wv[1], wsem[1])

    def chunk_step(c, p):
        drain_gathers(p)

        @pl.when(c + 2 < NCH)
        def _():
            pltpu.async_copy(idx_slice(c + 2), idxv[p], isem[p])

        @pl.when(c + 1 < NCH)
        def _():
            pltpu.make_async_copy(idx_slice(c + 1), idxv[1 - p], isem[1 - p]).wait()
            fire_gathers(1 - p, c + 1)

        @pl.when(c >= 2)
        def _():
            pltpu.make_async_copy(outv[p], out_slice(c - 2), osem[p]).wait()

        pltpu.make_async_copy(w_slice(c), wv[p], wsem[p]).wait()
        compute(p, c)
        pltpu.async_copy(outv[p], out_slice(c), osem[p])

        @pl.when(c + 2 < NCH)
        def _():
            pltpu.async_copy(w_slice(c + 2), wv[p], wsem[p])

    def pair(k, _):
        chunk_step(2 * k, 0)
        chunk_step(2 * k + 1, 1)
        return 0

    lax.fori_loop(0, NCH // 2, pair, 0)
    pltpu.make_async_copy(outv[0], out_slice(NCH - 2), osem[0]).wait()
    pltpu.make_async_copy(outv[1], out_slice(NCH - 1), osem[1]).wait()


def _bag_call(idx, w, table):
    mesh = plsc.VectorSubcoreMesh(core_axis_name="c", subcore_axis_name="s")
    f = pl.kernel(
        _bag_body,
        out_type=jax.ShapeDtypeStruct((NQH * DH,), jnp.float32),
        mesh=mesh,
        scratch_types=[
            pltpu.VMEM((CROWS,), jnp.int32), pltpu.VMEM((CROWS,), jnp.int32),
            pltpu.VMEM((CROWS,), jnp.float32), pltpu.VMEM((CROWS,), jnp.float32),
            pltpu.VMEM((CROWS, DH), jnp.float32), pltpu.VMEM((CROWS, DH), jnp.float32),
            pltpu.VMEM((CH * DH,), jnp.float32), pltpu.VMEM((CH * DH,), jnp.float32),
        ] + [pltpu.SemaphoreType.DMA] * 8,
        compiler_params=pltpu.CompilerParams(use_tc_tiling_on_sc=False),
    )
    return f(idx.reshape(NQH * CONTRIB), w.reshape(NQH * CONTRIB), table)


def _final_body(agg_ref, cnt_ref, q_ref, wdo_ref, bdo_ref, wo_ref, bo_ref, out_ref):
    cnt = cnt_ref[...]
    has = (cnt > 0.0).astype(jnp.float32)
    x = agg_ref[...] / jnp.maximum(cnt, 1.0)
    s = jnp.dot(x, wdo_ref[...], preferred_element_type=jnp.float32) + has * bdo_ref[...]
    o = jnp.dot(s, wo_ref[...], preferred_element_type=jnp.float32) + bo_ref[...]
    out_ref[...] = o + q_ref[...]


def _final_call(agg, cnt, q2, W_do, b_do, W_o, b_o):
    blk = 2000
    return pl.pallas_call(
        _final_body,
        grid=(NQ // blk,),
        in_specs=[
            pl.BlockSpec((blk, EMBED), lambda i: (i, 0)),
            pl.BlockSpec((blk, 1), lambda i: (i, 0)),
            pl.BlockSpec((blk, EMBED), lambda i: (i, 0)),
            pl.BlockSpec((EMBED, EMBED), lambda i: (0, 0)),
            pl.BlockSpec((1, EMBED), lambda i: (0, 0)),
            pl.BlockSpec((EMBED, EMBED), lambda i: (0, 0)),
            pl.BlockSpec((1, EMBED), lambda i: (0, 0)),
        ],
        out_specs=pl.BlockSpec((blk, EMBED), lambda i: (i, 0)),
        out_shape=jax.ShapeDtypeStruct((NQ, EMBED), jnp.float32),
    )(agg, cnt, q2, W_do, b_do[None, :], W_o, b_o[None, :])


def kernel(query, key, value, reference_points, spatial_shapes, reference_points_cam,
           bev_mask, W_off, b_off, W_attn, b_attn, W_v, b_v, W_do, b_do, W_o, b_o):
    del key, reference_points, spatial_shapes
    q2 = query[0]
    active = (bev_mask[:, 0].sum(-1) > 0).astype(jnp.float32)
    rcam = reference_points_cam[:, 0]

    table = _vp_call(value.reshape(CAMS * NK, EMBED), W_v, b_v).reshape(NROWS, DH)
    off, aw = _qproj_call(q2, W_off, b_off, W_attn, b_attn)
    off = off.reshape(NQ, HEADS, LEVELS, POINTS, 2)
    aw = aw.reshape(NQ, HEADS, LEVELS, POINTS)

    idx, w = _build_idx_w(off, aw, rcam, active)
    agg = _bag_call(idx, w, table).reshape(NQ, EMBED)

    cnt = active.sum(0)[:, None]
    out = _final_call(agg, cnt, q2, W_do, b_do, W_o, b_o)
    return out[None]


# speedup vs baseline: 5370.2269x; 6.5195x over previous
import functools

import jax
import jax.numpy as jnp
from jax import lax
from jax.experimental import pallas as pl
from jax.experimental.pallas import tpu as pltpu
from jax.experimental.pallas import tpu_sc as plsc

EMBED = 256
HEADS = 8
DH = EMBED // HEADS
LEVELS = 4
POINTS = 4
CAMS = 6
NQ = 10000
SHAPES = [(46, 80), (23, 40), (12, 20), (6, 10)]
NK = sum(h * w for h, w in SHAPES)
STARTS = [0, 3680, 4600, 4840]
NQH = NQ * HEADS
CONTRIB = CAMS * LEVELS * POINTS * 4
NROWS = CAMS * NK * HEADS

NTILES = 32
QH_T = NQH // NTILES
CH = 2
CROWS = CH * CONTRIB
NSUB = CROWS // 128
NCH = QH_T // CH


def _vp_body(v_ref, w_ref, b_ref, out_ref):
    out_ref[...] = (jnp.dot(v_ref[...], w_ref[...],
                            preferred_element_type=jnp.float32) + b_ref[...])


def _vp_call(val2d, W_v, b_v):
    blk = 4200
    return pl.pallas_call(
        _vp_body,
        grid=(val2d.shape[0] // blk,),
        in_specs=[
            pl.BlockSpec((blk, EMBED), lambda i: (i, 0)),
            pl.BlockSpec((EMBED, EMBED), lambda i: (0, 0)),
            pl.BlockSpec((1, EMBED), lambda i: (0, 0)),
        ],
        out_specs=pl.BlockSpec((blk, EMBED), lambda i: (i, 0)),
        out_shape=jax.ShapeDtypeStruct((val2d.shape[0], EMBED), jnp.float32),
    )(val2d, W_v, b_v[None, :])


def _qproj_body(q_ref, woffx_ref, boffx_ref, woffy_ref, boffy_ref,
                wattn_ref, battn_ref, offx_ref, offy_ref, aw_ref):
    q = q_ref[...]
    offx_ref[...] = (jnp.dot(q, woffx_ref[...], preferred_element_type=jnp.float32)
                     + boffx_ref[...])
    offy_ref[...] = (jnp.dot(q, woffy_ref[...], preferred_element_type=jnp.float32)
                     + boffy_ref[...])
    lg = (jnp.dot(q, wattn_ref[...], preferred_element_type=jnp.float32)
          + battn_ref[...])
    lane = lax.broadcasted_iota(jnp.int32, lg.shape, 1)
    m = lg
    for sh in (8, 4, 2, 1):
        lo = (lane & sh) == 0
        m = jnp.maximum(m, jnp.where(lo, pltpu.roll(m, 128 - sh, axis=1),
                                     pltpu.roll(m, sh, axis=1)))
    e = jnp.exp(lg - m)
    s = e
    for sh in (8, 4, 2, 1):
        lo = (lane & sh) == 0
        s = s + jnp.where(lo, pltpu.roll(s, 128 - sh, axis=1),
                          pltpu.roll(s, sh, axis=1))
    aw_ref[...] = e / s


def _qproj_call(q2, W_off, b_off, W_attn, b_attn):
    blk = 2000
    return pl.pallas_call(
        _qproj_body,
        grid=(NQ // blk,),
        in_specs=[
            pl.BlockSpec((blk, EMBED), lambda i: (i, 0)),
            pl.BlockSpec((EMBED, 128), lambda i: (0, 0)),
            pl.BlockSpec((1, 128), lambda i: (0, 0)),
            pl.BlockSpec((EMBED, 128), lambda i: (0, 0)),
            pl.BlockSpec((1, 128), lambda i: (0, 0)),
            pl.BlockSpec((EMBED, 128), lambda i: (0, 0)),
            pl.BlockSpec((1, 128), lambda i: (0, 0)),
        ],
        out_specs=[
            pl.BlockSpec((blk, 128), lambda i: (i, 0)),
            pl.BlockSpec((blk, 128), lambda i: (i, 0)),
            pl.BlockSpec((blk, 128), lambda i: (i, 0)),
        ],
        out_shape=[
            jax.ShapeDtypeStruct((NQ, 128), jnp.float32),
            jax.ShapeDtypeStruct((NQ, 128), jnp.float32),
            jax.ShapeDtypeStruct((NQ, 128), jnp.float32),
        ],
    )(q2, W_off[:, 0::2], b_off[None, 0::2], W_off[:, 1::2], b_off[None, 1::2],
      W_attn, b_attn[None, :])


def _idxw_body(offx_ref, offy_ref, aw_ref, rcam_ref, act_ref, idx_ref, w_ref):
    h = pl.program_id(1)
    blkq = offx_ref.shape[0]
    shp = (blkq, CONTRIB)
    lane = lax.broadcasted_iota(jnp.int32, shp, 1)
    cI = lane // (LEVELS * POINTS * 4)
    lI = (lane // (POINTS * 4)) % LEVELS
    pI = (lane // 4) % POINTS
    rI = lane % 4
    dx = (rI & 1).astype(jnp.float32)
    dy = (rI >> 1).astype(jnp.float32)

    def sel_by_level(vals):
        x = jnp.full(shp, vals[3], jnp.float32)
        x = jnp.where(lI == 0, vals[0], x)
        x = jnp.where(lI == 1, vals[1], x)
        x = jnp.where(lI == 2, vals[2], x)
        return x

    Wl = sel_by_level([float(w) for (hh, w) in SHAPES])
    Hl = sel_by_level([float(hh) for (hh, w) in SHAPES])
    start = sel_by_level([float(s) for s in STARTS])

    offx_h = offx_ref[...]
    offy_h = offy_ref[...]
    aw_h = aw_ref[...]
    rc = rcam_ref[...]
    act = act_ref[...]

    o_idx = h * 16 + lI * POINTS + pI
    of_x = jnp.take_along_axis(offx_h, o_idx, axis=1)
    of_y = jnp.take_along_axis(offy_h, o_idx, axis=1)
    awl = jnp.take_along_axis(aw_h, o_idx, axis=1)
    r_idx = cI * (LEVELS * 2) + lI * 2
    rc_x = jnp.take_along_axis(rc, r_idx, axis=1)
    rc_y = jnp.take_along_axis(rc, r_idx + 1, axis=1)
    actc = jnp.take_along_axis(act, cI, axis=1)

    x = rc_x * Wl + of_x - 0.5
    y = rc_y * Hl + of_y - 0.5
    x0 = jnp.floor(x)
    y0 = jnp.floor(y)
    wx1 = x - x0
    wy1 = y - y0
    xi = x0 + dx
    yi = y0 + dy
    valid = ((xi >= 0) & (xi <= Wl - 1) & (yi >= 0) & (yi <= Hl - 1))
    xc = jnp.clip(xi, 0, Wl - 1)
    yc = jnp.clip(yi, 0, Hl - 1)
    pix = start + yc * Wl + xc
    row = (cI * NK) * HEADS + pix.astype(jnp.int32) * HEADS + h
    cw = (jnp.where((rI & 1) != 0, wx1, 1.0 - wx1)
          * jnp.where((rI >> 1) != 0, wy1, 1.0 - wy1))
    idx_ref[...] = row
    w_ref[...] = awl * actc * cw * valid.astype(jnp.float32)


def _build_idx_w(offx, offy, aw2d, rcam48, act8):
    blkq = 1000
    return pl.pallas_call(
        _idxw_body,
        grid=(NQ // blkq, HEADS),
        in_specs=[
            pl.BlockSpec((blkq, 128), lambda i, h: (i, 0)),
            pl.BlockSpec((blkq, 128), lambda i, h: (i, 0)),
            pl.BlockSpec((blkq, 128), lambda i, h: (i, 0)),
            pl.BlockSpec((blkq, 48), lambda i, h: (i, 0)),
            pl.BlockSpec((blkq, 8), lambda i, h: (i, 0)),
        ],
        out_specs=[
            pl.BlockSpec((blkq, CONTRIB), lambda i, h: (i, h)),
            pl.BlockSpec((blkq, CONTRIB), lambda i, h: (i, h)),
        ],
        out_shape=[
            jax.ShapeDtypeStruct((NQ, HEADS * CONTRIB), jnp.int32),
            jax.ShapeDtypeStruct((NQ, HEADS * CONTRIB), jnp.float32),
        ],
    )(offx, offy, aw2d, rcam48, act8)


def _bag_body(idx_hbm, w_hbm, table_hbm, out_hbm,
              idxv0, idxv1, wv0, wv1, rows0, rows1, outv0, outv1,
              g0, g1, i0, i1, ws0, ws1, o0, o1):
    idxv = (idxv0, idxv1)
    wv = (wv0, wv1)
    rows = (rows0, rows1)
    outv = (outv0, outv1)
    gsem = (g0, g1)
    isem = (i0, i1)
    wsem = (ws0, ws1)
    osem = (o0, o1)

    wid = lax.axis_index("s") * 2 + lax.axis_index("c")
    qh0 = wid * QH_T

    def idx_slice(c):
        return idx_hbm.at[pl.ds((qh0 + c * CH) * CONTRIB, CROWS)]

    def w_slice(c):
        return w_hbm.at[pl.ds((qh0 + c * CH) * CONTRIB, CROWS)]

    def out_slice(c):
        return out_hbm.at[pl.ds((qh0 + c * CH) * DH, CH * DH)]

    def fire_gathers(p, c):
        for s in range(NSUB):
            pltpu.async_copy(
                table_hbm.at[idxv[p].at[pl.ds(s * 128, 128)]],
                rows[p].at[pl.ds(s * 128, 128), :], gsem[p])

    def drain_gathers(p):
        for s in range(NSUB):
            pltpu.make_async_copy(
                table_hbm.at[idxv[p].at[pl.ds(s * 128, 128)]],
                rows[p].at[pl.ds(s * 128, 128), :], gsem[p]).wait()

    iota16 = lax.broadcasted_iota(jnp.int32, (16,), 0)

    def compute(p, c):
        for j in range(CH):
            jbase = j * CONTRIB

            def g_body(g, accs):
                acc0, acc1 = accs
                k0 = jbase + g * 16
                wv16 = wv[p][pl.ds(k0, 16)]
                for t in range(16):
                    wsplat = jnp.take_along_axis(
                        wv16, jnp.full((16,), t, jnp.int32), axis=0)
                    r0 = rows[p][k0 + t, pl.ds(0, 16)]
                    r1 = rows[p][k0 + t, pl.ds(16, 16)]
                    acc0 = acc0 + wsplat * r0
                    acc1 = acc1 + wsplat * r1
                return acc0, acc1

            z = jnp.zeros((16,), jnp.float32)
            acc0, acc1 = lax.fori_loop(0, CONTRIB // 16, g_body, (z, z))
            outv[p][pl.ds(j * DH, 16)] = acc0
            outv[p][pl.ds(j * DH + 16, 16)] = acc1

    pltpu.sync_copy(idx_slice(0), idxv[0])
    fire_gathers(0, 0)
    pltpu.async_copy(idx_slice(1), idxv[1], isem[1])
    pltpu.async_copy(w_slice(0), wv[0], wsem[0])
    pltpu.async_copy(w_slice(1), wv[1], wsem[1])

    def chunk_step(c, p):
        drain_gathers(p)

        @pl.when(c + 2 < NCH)
        def _():
            pltpu.async_copy(idx_slice(c + 2), idxv[p], isem[p])

        @pl.when(c + 1 < NCH)
        def _():
            pltpu.make_async_copy(idx_slice(c + 1), idxv[1 - p], isem[1 - p]).wait()
            fire_gathers(1 - p, c + 1)

        @pl.when(c >= 2)
        def _():
            pltpu.make_async_copy(outv[p], out_slice(c - 2), osem[p]).wait()

        pltpu.make_async_copy(w_slice(c), wv[p], wsem[p]).wait()
        compute(p, c)
        pltpu.async_copy(outv[p], out_slice(c), osem[p])

        @pl.when(c + 2 < NCH)
        def _():
            pltpu.async_copy(w_slice(c + 2), wv[p], wsem[p])

    def pair(k, _):
        chunk_step(2 * k, 0)
        chunk_step(2 * k + 1, 1)
        return 0

    lax.fori_loop(0, NCH // 2, pair, 0)
    pltpu.make_async_copy(outv[0], out_slice(NCH - 2), osem[0]).wait()
    pltpu.make_async_copy(outv[1], out_slice(NCH - 1), osem[1]).wait()


def _bag_call(idx, w, table):
    mesh = plsc.VectorSubcoreMesh(core_axis_name="c", subcore_axis_name="s")
    f = pl.kernel(
        _bag_body,
        out_type=jax.ShapeDtypeStruct((NQH * DH,), jnp.float32),
        mesh=mesh,
        scratch_types=[
            pltpu.VMEM((CROWS,), jnp.int32), pltpu.VMEM((CROWS,), jnp.int32),
            pltpu.VMEM((CROWS,), jnp.float32), pltpu.VMEM((CROWS,), jnp.float32),
            pltpu.VMEM((CROWS, DH), jnp.float32), pltpu.VMEM((CROWS, DH), jnp.float32),
            pltpu.VMEM((CH * DH,), jnp.float32), pltpu.VMEM((CH * DH,), jnp.float32),
        ] + [pltpu.SemaphoreType.DMA] * 8,
        compiler_params=pltpu.CompilerParams(use_tc_tiling_on_sc=False),
    )
    return f(idx.reshape(NQH * CONTRIB), w.reshape(NQH * CONTRIB), table)


def _final_body(agg_ref, cnt_ref, q_ref, wdo_ref, bdo_ref, wo_ref, bo_ref, out_ref):
    cnt = cnt_ref[...]
    has = (cnt > 0.0).astype(jnp.float32)
    x = agg_ref[...] / jnp.maximum(cnt, 1.0)
    s = jnp.dot(x, wdo_ref[...], preferred_element_type=jnp.float32) + has * bdo_ref[...]
    o = jnp.dot(s, wo_ref[...], preferred_element_type=jnp.float32) + bo_ref[...]
    out_ref[...] = o + q_ref[...]


def _final_call(agg, cnt, q2, W_do, b_do, W_o, b_o):
    blk = 2000
    return pl.pallas_call(
        _final_body,
        grid=(NQ // blk,),
        in_specs=[
            pl.BlockSpec((blk, EMBED), lambda i: (i, 0)),
            pl.BlockSpec((blk, 1), lambda i: (i, 0)),
            pl.BlockSpec((blk, EMBED), lambda i: (i, 0)),
            pl.BlockSpec((EMBED, EMBED), lambda i: (0, 0)),
            pl.BlockSpec((1, EMBED), lambda i: (0, 0)),
            pl.BlockSpec((EMBED, EMBED), lambda i: (0, 0)),
            pl.BlockSpec((1, EMBED), lambda i: (0, 0)),
        ],
        out_specs=pl.BlockSpec((blk, EMBED), lambda i: (i, 0)),
        out_shape=jax.ShapeDtypeStruct((NQ, EMBED), jnp.float32),
    )(agg, cnt, q2, W_do, b_do[None, :], W_o, b_o[None, :])


def kernel(query, key, value, reference_points, spatial_shapes, reference_points_cam,
           bev_mask, W_off, b_off, W_attn, b_attn, W_v, b_v, W_do, b_do, W_o, b_o):
    del key, reference_points, spatial_shapes
    q2 = query[0]
    active = (bev_mask[:, 0].sum(-1) > 0).astype(jnp.float32)
    rcam = reference_points_cam[:, 0]

    table = _vp_call(value.reshape(CAMS * NK, EMBED), W_v, b_v).reshape(NROWS, DH)
    offx, offy, aw = _qproj_call(q2, W_off, b_off, W_attn, b_attn)

    rcam48 = rcam.transpose(1, 0, 2, 3).reshape(NQ, CAMS * LEVELS * 2)
    act8 = jnp.pad(active.T, ((0, 0), (0, 8 - CAMS)))
    idx, w = _build_idx_w(offx, offy, aw, rcam48, act8)
    agg = _bag_call(idx, w, table).reshape(NQ, EMBED)

    cnt = active.sum(0)[:, None]
    out = _final_call(agg, cnt, q2, W_do, b_do, W_o, b_o)
    return out[None]
